# baseline (speedup 1.0000x reference)
import jax
import jax.numpy as jnp
from jax import lax
from jax.experimental import pallas as pl
from jax.experimental.pallas import tpu as pltpu
from jax.experimental.pallas import tpu_sc as plsc

_NUM_SCHEMAS = 16
_K_ACTIVE = 3
_BJ = 512
_BK = 2048
_NC = 1
_BIAS_CHUNK = 128


def _sc_select_body(row_hbm, bias_hbm, idx_hbm, w_hbm, selbias_hbm,
                    rowbuf, idxbuf, wbuf, biasbuf, accbuf):
    wid = lax.axis_index("s") * _NC + lax.axis_index("c")

    pltpu.sync_copy(row_hbm, rowbuf)
    row = rowbuf[...]
    iota = lax.iota(jnp.int32, _NUM_SCHEMAS)
    vals = row
    keepf = jnp.zeros((_NUM_SCHEMAS,), jnp.float32)
    idxvec = jnp.zeros((_NUM_SCHEMAS,), jnp.int32)
    wvec = jnp.zeros((_NUM_SCHEMAS,), jnp.float32)
    for a in range(_K_ACTIVE):
        m = jnp.max(vals)
        am = jnp.min(jnp.where(vals == m, iota, _NUM_SCHEMAS))
        idxvec = jnp.where(iota == a, am, idxvec)
        wvec = jnp.where(iota == a, m, wvec)
        hit = iota == am
        keepf = jnp.where(hit, 1.0, keepf)
        vals = jnp.where(hit, jnp.float32(-jnp.inf), vals)
    wfull = row * keepf

    @pl.when(wid == 0)
    def _publish():
        idxbuf[...] = idxvec
        wbuf[...] = wvec
        pltpu.sync_copy(idxbuf, idx_hbm)
        pltpu.sync_copy(wbuf, w_hbm)

    @pl.when(wid < selbias_hbm.shape[0] // _BIAS_CHUNK)
    def _bias():
        pltpu.sync_copy(bias_hbm.at[:, pl.ds(wid * _BIAS_CHUNK, _BIAS_CHUNK)],
                        biasbuf)
        accs = [jnp.zeros((16,), jnp.float32)
                for _ in range(_BIAS_CHUNK // 16)]
        for i in range(_NUM_SCHEMAS):
            w_i = wfull[i]
            for c in range(_BIAS_CHUNK // 16):
                accs[c] = accs[c] + w_i * biasbuf[i, pl.ds(c * 16, 16)]
        for c in range(_BIAS_CHUNK // 16):
            accbuf[pl.ds(c * 16, 16)] = accs[c]
        pltpu.sync_copy(accbuf,
                        selbias_hbm.at[pl.ds(wid * _BIAS_CHUNK, _BIAS_CHUNK)])


def _combine_body(idx_ref, aw0_ref, aw1_ref, aw2_ref, w_ref, out_ref):
    out_ref[...] = (w_ref[0] * aw0_ref[0].T
                    + w_ref[1] * aw1_ref[0].T
                    + w_ref[2] * aw2_ref[0].T)


def kernel(all_weight, all_bias, schema_weighting, unit_idx):
    n_schemas, c_in, c_out = all_weight.shape
    row = jnp.take(schema_weighting, jnp.asarray(unit_idx, jnp.int32), axis=0)

    select = pl.kernel(
        _sc_select_body,
        out_type=(
            jax.ShapeDtypeStruct((_NUM_SCHEMAS,), jnp.int32),
            jax.ShapeDtypeStruct((_NUM_SCHEMAS,), jnp.float32),
            jax.ShapeDtypeStruct((c_out,), jnp.float32),
        ),
        mesh=plsc.VectorSubcoreMesh(core_axis_name="c", subcore_axis_name="s",
                                    num_cores=_NC),
        compiler_params=pltpu.CompilerParams(needs_layout_passes=False,
                                             skip_device_barrier=True),
        scratch_types=[
            pltpu.VMEM((_NUM_SCHEMAS,), jnp.float32),
            pltpu.VMEM((_NUM_SCHEMAS,), jnp.int32),
            pltpu.VMEM((_NUM_SCHEMAS,), jnp.float32),
            pltpu.VMEM((_NUM_SCHEMAS, _BIAS_CHUNK), jnp.float32),
            pltpu.VMEM((_BIAS_CHUNK,), jnp.float32),
        ],
    )
    idx16, w16, selbias = select(row, all_bias)

    nk = c_out // _BK
    nj = c_in // _BJ
    grid_spec = pltpu.PrefetchScalarGridSpec(
        num_scalar_prefetch=1,
        grid=(nk, nj),
        in_specs=[
            pl.BlockSpec((1, _BJ, _BK), lambda k, j, idx_ref: (idx_ref[0], j, k)),
            pl.BlockSpec((1, _BJ, _BK), lambda k, j, idx_ref: (idx_ref[1], j, k)),
            pl.BlockSpec((1, _BJ, _BK), lambda k, j, idx_ref: (idx_ref[2], j, k)),
            pl.BlockSpec(memory_space=pltpu.SMEM),
        ],
        out_specs=pl.BlockSpec((_BK, _BJ), lambda k, j, idx_ref: (k, j)),
    )
    sel_weight = pl.pallas_call(
        _combine_body,
        grid_spec=grid_spec,
        out_shape=jax.ShapeDtypeStruct((c_out, c_in), jnp.float32),
    )(idx16, all_weight, all_weight, all_weight, w16)

    return sel_weight, selbias


# speedup vs baseline: 1.0176x; 1.0176x over previous
import jax
import jax.numpy as jnp
from jax import lax
from jax.experimental import pallas as pl
from jax.experimental.pallas import tpu as pltpu
from jax.experimental.pallas import tpu_sc as plsc

_NUM_SCHEMAS = 16
_K_ACTIVE = 3
_BJ = 512
_BK = 2048
_NC = 1
_NS = 16
_BIAS_CHUNK = 128


def _select_body(unit_ref, sw_ref, idx_ref, w_ref):
    u = unit_ref[0]
    row = sw_ref[pl.ds(u, 1), :]
    iota = jax.lax.broadcasted_iota(jnp.int32, (1, _NUM_SCHEMAS), 1)
    vals = row
    for a in range(_K_ACTIVE):
        m = jnp.max(vals)
        am = jnp.min(jnp.where(vals == m, iota, _NUM_SCHEMAS)).astype(jnp.int32)
        idx_ref[a] = am
        w_ref[a] = m
        vals = jnp.where(iota == am, -jnp.inf, vals)


def _sc_bias_body(row_hbm, bias_hbm, selbias_hbm, rowbuf, biasbuf, accbuf):
    wid = lax.axis_index("s") * _NC + lax.axis_index("c")

    @pl.when(wid < selbias_hbm.shape[0] // _BIAS_CHUNK)
    def _bias():
        pltpu.sync_copy(row_hbm, rowbuf)
        row = rowbuf[...]
        iota = lax.iota(jnp.int32, _NUM_SCHEMAS)
        vals = row
        keepf = jnp.zeros((_NUM_SCHEMAS,), jnp.float32)
        for a in range(_K_ACTIVE):
            m = jnp.max(vals)
            am = jnp.min(jnp.where(vals == m, iota, _NUM_SCHEMAS))
            hit = iota == am
            keepf = jnp.where(hit, 1.0, keepf)
            vals = jnp.where(hit, jnp.float32(-jnp.inf), vals)
        wfull = row * keepf

        pltpu.sync_copy(bias_hbm.at[:, pl.ds(wid * _BIAS_CHUNK, _BIAS_CHUNK)],
                        biasbuf)
        accs = [jnp.zeros((16,), jnp.float32)
                for _ in range(_BIAS_CHUNK // 16)]
        for i in range(_NUM_SCHEMAS):
            w_i = wfull[i]
            for c in range(_BIAS_CHUNK // 16):
                accs[c] = accs[c] + w_i * biasbuf[i, pl.ds(c * 16, 16)]
        for c in range(_BIAS_CHUNK // 16):
            accbuf[pl.ds(c * 16, 16)] = accs[c]
        pltpu.sync_copy(accbuf,
                        selbias_hbm.at[pl.ds(wid * _BIAS_CHUNK, _BIAS_CHUNK)])


def _combine_body(idx_ref, aw0_ref, aw1_ref, aw2_ref, w_ref, out_ref):
    out_ref[...] = (w_ref[0] * aw0_ref[0].T
                    + w_ref[1] * aw1_ref[0].T
                    + w_ref[2] * aw2_ref[0].T)


def kernel(all_weight, all_bias, schema_weighting, unit_idx):
    n_schemas, c_in, c_out = all_weight.shape
    unit = jnp.asarray(unit_idx, jnp.int32).reshape((1,))
    row = jnp.take(schema_weighting, jnp.asarray(unit_idx, jnp.int32), axis=0)

    sc_bias = pl.kernel(
        _sc_bias_body,
        out_type=jax.ShapeDtypeStruct((c_out,), jnp.float32),
        mesh=plsc.VectorSubcoreMesh(core_axis_name="c", subcore_axis_name="s",
                                    num_cores=_NC),
        compiler_params=pltpu.CompilerParams(needs_layout_passes=False,
                                             skip_device_barrier=True),
        scratch_types=[
            pltpu.VMEM((_NUM_SCHEMAS,), jnp.float32),
            pltpu.VMEM((_NUM_SCHEMAS, _BIAS_CHUNK), jnp.float32),
            pltpu.VMEM((_BIAS_CHUNK,), jnp.float32),
        ],
    )
    idx, w = pl.pallas_call(
        _select_body,
        in_specs=[
            pl.BlockSpec(memory_space=pltpu.SMEM),
            pl.BlockSpec(memory_space=pltpu.VMEM),
        ],
        out_specs=[
            pl.BlockSpec(memory_space=pltpu.SMEM),
            pl.BlockSpec(memory_space=pltpu.SMEM),
        ],
        out_shape=[
            jax.ShapeDtypeStruct((_K_ACTIVE,), jnp.int32),
            jax.ShapeDtypeStruct((_K_ACTIVE,), jnp.float32),
        ],
    )(unit, schema_weighting)

    nk = c_out // _BK
    nj = c_in // _BJ
    grid_spec = pltpu.PrefetchScalarGridSpec(
        num_scalar_prefetch=1,
        grid=(nk, nj),
        in_specs=[
            pl.BlockSpec((1, _BJ, _BK), lambda k, j, idx_ref: (idx_ref[0], j, k)),
            pl.BlockSpec((1, _BJ, _BK), lambda k, j, idx_ref: (idx_ref[1], j, k)),
            pl.BlockSpec((1, _BJ, _BK), lambda k, j, idx_ref: (idx_ref[2], j, k)),
            pl.BlockSpec(memory_space=pltpu.SMEM),
        ],
        out_specs=pl.BlockSpec((_BK, _BJ), lambda k, j, idx_ref: (k, j)),
    )
    sel_weight = pl.pallas_call(
        _combine_body,
        grid_spec=grid_spec,
        out_shape=jax.ShapeDtypeStruct((c_out, c_in), jnp.float32),
    )(idx, all_weight, all_weight, all_weight, w)

    selbias = sc_bias(row, all_bias)

    return sel_weight, selbias
